# baseline (speedup 1.0000x reference)
import functools

import jax
import jax.numpy as jnp
from jax import lax
from jax.experimental import pallas as pl
from jax.experimental.pallas import tpu as pltpu
from jax.experimental.pallas import tpu_sc as plsc

T, C, H = 2048, 1024, 4096
E, TOPK = 8, 2
CAP = 320
STRIDE = 352
NSLOT = E * STRIDE
SENTINEL = CAP

NC, NS = 2, 16
NW = NC * NS
SLOT_PER_W = NSLOT // NW
TOK_PER_W = T // NW


def _dot(a, b):
    return jax.lax.dot_general(a, b, (((1,), (0,)), ((), ())),
                               preferred_element_type=jnp.float32)


def _mm_kernel(a_ref, b_ref, bias_ref, o_ref, *, relu):
    acc = _dot(a_ref[...], b_ref[...]) + bias_ref[...]
    o_ref[...] = jnp.maximum(acc, 0.0) if relu else acc


def _mm_resident(a, b, bias, relu, nt):
    M, K = a.shape
    _, N = b.shape
    return pl.pallas_call(
        functools.partial(_mm_kernel, relu=relu),
        grid=(N // nt,),
        in_specs=[
            pl.BlockSpec((M, K), lambda j: (0, 0)),
            pl.BlockSpec((K, nt), lambda j: (0, j)),
            pl.BlockSpec((1, nt), lambda j: (0, j)),
        ],
        out_specs=pl.BlockSpec((M, nt), lambda j: (0, j)),
        out_shape=jax.ShapeDtypeStruct((M, N), jnp.float32),
        compiler_params=pltpu.CompilerParams(
            dimension_semantics=("arbitrary",)),
    )(a, b, bias.reshape(1, -1))


def _mm_stream_a(a, b, bias, relu, mt):
    M, K = a.shape
    _, N = b.shape
    return pl.pallas_call(
        functools.partial(_mm_kernel, relu=relu),
        grid=(M // mt,),
        in_specs=[
            pl.BlockSpec((mt, K), lambda i: (i, 0)),
            pl.BlockSpec((K, N), lambda i: (0, 0)),
            pl.BlockSpec((1, N), lambda i: (0, 0)),
        ],
        out_specs=pl.BlockSpec((mt, N), lambda i: (i, 0)),
        out_shape=jax.ShapeDtypeStruct((M, N), jnp.float32),
        compiler_params=pltpu.CompilerParams(
            dimension_semantics=("arbitrary",)),
    )(a, b, bias.reshape(1, -1))


def _dispatch_body(logits, slot0_ref, slot1_ref, wslot_ref, tok_ref,
                   cum_ref, a_ref):
    lane = jax.lax.broadcasted_iota(jnp.int32, (T, E), 1)
    m = jnp.max(logits, axis=1, keepdims=True)
    ex = jnp.exp(logits - m)
    probs = ex / jnp.sum(ex, axis=1, keepdims=True)

    p0 = jnp.max(probs, axis=1, keepdims=True)
    e0 = jnp.min(jnp.where(probs == p0, lane, E), axis=1, keepdims=True)
    pm = jnp.where(lane == e0, -1.0, probs)
    p1 = jnp.max(pm, axis=1, keepdims=True)
    e1 = jnp.min(jnp.where(pm == p1, lane, E), axis=1, keepdims=True)

    oh0 = (lane == e0).astype(jnp.float32)
    oh1 = (lane == e1).astype(jnp.float32)
    a_ref[...] = oh0 + oh1

    row = jax.lax.broadcasted_iota(jnp.int32, (128, 128), 0)
    col = jax.lax.broadcasted_iota(jnp.int32, (128, 128), 1)
    lstrict = (col < row).astype(jnp.float32)

    def body(i, carry):
        ablk = a_ref[pl.ds(i * 128, 128), :]
        cum_ref[pl.ds(i * 128, 128), :] = carry + _dot(lstrict, ablk)
        return carry + jnp.sum(ablk, axis=0, keepdims=True)

    jax.lax.fori_loop(0, T // 128, body, jnp.zeros((1, E), jnp.float32))
    cum = cum_ref[...]

    r0 = jnp.sum(cum * oh0, axis=1, keepdims=True)
    r1 = jnp.sum(cum * oh1, axis=1, keepdims=True)
    kept0 = r0 < float(CAP)
    kept1 = r1 < float(CAP)
    fs0 = e0.astype(jnp.float32) * STRIDE + r0
    fs1 = e1.astype(jnp.float32) * STRIDE + r1
    s0 = jnp.where(kept0, fs0, float(SENTINEL)).astype(jnp.int32)
    s1 = jnp.where(kept1, fs1, float(SENTINEL)).astype(jnp.int32)
    w0 = jnp.where(kept0, p0, 0.0)
    w1 = jnp.where(kept1, p1, 0.0)
    slot0_ref[...] = s0
    slot1_ref[...] = s1

    tf = jax.lax.broadcasted_iota(jnp.int32, (T, 1), 0).astype(jnp.float32)
    vt0 = jnp.where(kept0, tf, 0.0)
    vt1 = jnp.where(kept1, tf, 0.0)

    def wbody(j, _):
        sidx = j * 128 + jax.lax.broadcasted_iota(jnp.int32, (T, 128), 1)
        hit0 = s0 == sidx
        hit1 = s1 == sidx
        m0 = jnp.where(hit0, w0, 0.0)
        m1 = jnp.where(hit1, w1, 0.0)
        wslot_ref[:, pl.ds(j * 128, 128)] = jnp.sum(m0 + m1, axis=0,
                                                    keepdims=True)
        t0 = jnp.where(hit0, vt0, 0.0)
        t1 = jnp.where(hit1, vt1, 0.0)
        tok_ref[:, pl.ds(j * 128, 128)] = jnp.sum(t0 + t1, axis=0,
                                                  keepdims=True).astype(jnp.int32)
        return 0

    jax.lax.fori_loop(0, NSLOT // 128, wbody, 0)


def _dispatch_kernel(logits_ref, slot0_ref, slot1_ref, wslot_ref, tok_ref,
                     cum_ref, a_ref):
    _dispatch_body(logits_ref[...], slot0_ref, slot1_ref, wslot_ref, tok_ref,
                   cum_ref, a_ref)


def _dispatch(logits):
    return pl.pallas_call(
        _dispatch_kernel,
        in_specs=[pl.BlockSpec((T, E), lambda: (0, 0))],
        out_specs=[
            pl.BlockSpec((T, 1), lambda: (0, 0)),
            pl.BlockSpec((T, 1), lambda: (0, 0)),
            pl.BlockSpec((1, NSLOT), lambda: (0, 0)),
            pl.BlockSpec((1, NSLOT), lambda: (0, 0)),
        ],
        out_shape=[
            jax.ShapeDtypeStruct((T, 1), jnp.int32),
            jax.ShapeDtypeStruct((T, 1), jnp.int32),
            jax.ShapeDtypeStruct((1, NSLOT), jnp.float32),
            jax.ShapeDtypeStruct((1, NSLOT), jnp.int32),
        ],
        scratch_shapes=[pltpu.VMEM((T, E), jnp.float32),
                        pltpu.VMEM((T, E), jnp.float32)],
    )(logits)


def _mm2_kernel(a_ref, b_ref, bias_ref, wr3_ref, br3_ref, lg_ref, *, nsteps):
    j = pl.program_id(0)
    h2 = jnp.maximum(_dot(a_ref[...], b_ref[...]) + bias_ref[...], 0.0)
    part = _dot(h2, wr3_ref[...])

    @pl.when(j == 0)
    def _():
        lg_ref[...] = part + br3_ref[...]

    @pl.when(j > 0)
    def _():
        lg_ref[...] += part


def _mm2_logits(h1, Wr2, br2, Wr3, br3, nt=256):
    grid = (H // nt,)
    return pl.pallas_call(
        functools.partial(_mm2_kernel, nsteps=grid[0]),
        grid=grid,
        in_specs=[
            pl.BlockSpec((T, H), lambda j: (0, 0)),
            pl.BlockSpec((H, nt), lambda j: (0, j)),
            pl.BlockSpec((1, nt), lambda j: (0, j)),
            pl.BlockSpec((nt, E), lambda j: (j, 0)),
            pl.BlockSpec((1, E), lambda j: (0, 0)),
        ],
        out_specs=pl.BlockSpec((T, E), lambda j: (0, 0)),
        out_shape=jax.ShapeDtypeStruct((T, E), jnp.float32),
        compiler_params=pltpu.CompilerParams(
            dimension_semantics=("arbitrary",)),
    )(h1, Wr2, br2.reshape(1, -1), Wr3, br3.reshape(1, -1))


_SC_MESH = plsc.VectorSubcoreMesh(core_axis_name="c", subcore_axis_name="s")


@functools.partial(
    pl.kernel, mesh=_SC_MESH,
    out_type=jax.ShapeDtypeStruct((NSLOT, C), jnp.float32),
    scratch_types=[
        pltpu.VMEM((SLOT_PER_W,), jnp.int32),
        pltpu.VMEM((SLOT_PER_W, C), jnp.float32),
        pltpu.SemaphoreType.DMA,
    ],
)
def _sc_gather(x_hbm, tok_hbm, xe_hbm, idx_v, rows_v, sem):
    wid = lax.axis_index("s") * NC + lax.axis_index("c")
    base = wid * SLOT_PER_W
    pltpu.sync_copy(tok_hbm.at[pl.ds(base, SLOT_PER_W)], idx_v)
    nchunk, csz = SLOT_PER_W // 8, 8
    copies = [
        pltpu.async_copy(x_hbm.at[idx_v.at[pl.ds(i * csz, csz)]],
                         rows_v.at[pl.ds(i * csz, csz)], sem)
        for i in range(nchunk)
    ]
    for cp in copies:
        cp.wait()
    pltpu.sync_copy(rows_v, xe_hbm.at[pl.ds(base, SLOT_PER_W)])


def _ffn_kernel(xe_ref, w1_ref, b1_ref, w2_ref, b2_ref, ws_ref, o_ref,
                *, nsteps):
    j = pl.program_id(1)
    hblk = jnp.maximum(_dot(xe_ref[...], w1_ref[0]) + b1_ref[0], 0.0)
    part = _dot(hblk, w2_ref[0])

    @pl.when(j == 0)
    def _():
        o_ref[...] = part

    @pl.when(j > 0)
    def _():
        o_ref[...] += part

    @pl.when(j == nsteps - 1)
    def _():
        o_ref[...] = (o_ref[...] + b2_ref[0]) * ws_ref[...]


def _ffn(xe, W1, b1, W2, b2, wslot_col, ht=1024):
    grid = (E, H // ht)
    return pl.pallas_call(
        functools.partial(_ffn_kernel, nsteps=grid[1]),
        grid=grid,
        in_specs=[
            pl.BlockSpec((STRIDE, C), lambda e, j: (e, 0)),
            pl.BlockSpec((1, C, ht), lambda e, j: (e, 0, j)),
            pl.BlockSpec((1, 1, ht), lambda e, j: (e, 0, j)),
            pl.BlockSpec((1, ht, C), lambda e, j: (e, j, 0)),
            pl.BlockSpec((1, 1, C), lambda e, j: (e, 0, 0)),
            pl.BlockSpec((STRIDE, 1), lambda e, j: (e, 0)),
        ],
        out_specs=pl.BlockSpec((STRIDE, C), lambda e, j: (e, 0)),
        out_shape=jax.ShapeDtypeStruct((NSLOT, C), jnp.float32),
        compiler_params=pltpu.CompilerParams(
            dimension_semantics=("arbitrary", "arbitrary")),
    )(xe, W1, b1.reshape(E, 1, H), W2, b2.reshape(E, 1, C), wslot_col)


@functools.partial(
    pl.kernel, mesh=_SC_MESH,
    out_type=jax.ShapeDtypeStruct((T, C), jnp.float32),
    scratch_types=[
        pltpu.VMEM((32,), jnp.int32),
        pltpu.VMEM((32,), jnp.int32),
        pltpu.VMEM((32, C), jnp.float32),
        pltpu.VMEM((32, C), jnp.float32),
        pltpu.SemaphoreType.DMA,
    ],
)
def _sc_combine(y_hbm, s0_hbm, s1_hbm, out_hbm, i0_v, i1_v, r0_v, r1_v, sem):
    wid = lax.axis_index("s") * NC + lax.axis_index("c")
    base = wid * TOK_PER_W
    for half in range(2):
        off = base + half * 32
        pltpu.sync_copy(s0_hbm.at[pl.ds(off, 32)], i0_v)
        pltpu.sync_copy(s1_hbm.at[pl.ds(off, 32)], i1_v)
        copies = [
            pltpu.async_copy(y_hbm.at[iv.at[pl.ds(i * 8, 8)]],
                             rv.at[pl.ds(i * 8, 8)], sem)
            for i in range(4) for iv, rv in ((i0_v, r0_v), (i1_v, r1_v))
        ]
        for cp in copies:
            cp.wait()
        for r in range(32):
            def add_body(cc, _):
                sl = pl.ds(cc * 16, 16)
                r0_v[r, sl] = r0_v[r, sl] + r1_v[r, sl]
                return 0
            lax.fori_loop(0, C // 16, add_body, 0, unroll=8)
        pltpu.sync_copy(r0_v, out_hbm.at[pl.ds(off, 32)])


def kernel(x, Wr1, br1, Wr2, br2, Wr3, br3, W1, b1, W2, b2):
    x2 = x.reshape(T, C)
    h1 = _mm_resident(x2, Wr1, br1, True, 512)
    logits = _mm2_logits(h1, Wr2, br2, Wr3, br3)
    slot0, slot1, wslot, tok = _dispatch(logits)
    xe = _sc_gather(x2, tok.reshape(NSLOT))
    Y = _ffn(xe, W1, b1, W2, b2, wslot.reshape(NSLOT, 1))
    out = _sc_combine(Y, slot0.reshape(T), slot1.reshape(T))
    return out.reshape(1, T, C)


# speedup vs baseline: 1.6129x; 1.6129x over previous
import functools

import jax
import jax.numpy as jnp
from jax import lax
from jax.experimental import pallas as pl
from jax.experimental.pallas import tpu as pltpu
from jax.experimental.pallas import tpu_sc as plsc

T, C, H = 2048, 1024, 4096
E, TOPK = 8, 2
CAP = 320
STRIDE = 352
NSLOT = E * STRIDE
SENTINEL = CAP

NC, NS = 2, 16
NW = NC * NS
SLOT_PER_W = NSLOT // NW
TOK_PER_W = T // NW


def _dot(a, b):
    return jax.lax.dot_general(a, b, (((1,), (0,)), ((), ())),
                               preferred_element_type=jnp.float32)


def _mm_kernel(a_ref, b_ref, bias_ref, o_ref, *, relu):
    acc = _dot(a_ref[...], b_ref[...]) + bias_ref[...]
    o_ref[...] = jnp.maximum(acc, 0.0) if relu else acc


def _mm_resident(a, b, bias, relu, nt):
    M, K = a.shape
    _, N = b.shape
    return pl.pallas_call(
        functools.partial(_mm_kernel, relu=relu),
        grid=(N // nt,),
        in_specs=[
            pl.BlockSpec((M, K), lambda j: (0, 0)),
            pl.BlockSpec((K, nt), lambda j: (0, j)),
            pl.BlockSpec((1, nt), lambda j: (0, j)),
        ],
        out_specs=pl.BlockSpec((M, nt), lambda j: (0, j)),
        out_shape=jax.ShapeDtypeStruct((M, N), jnp.float32),
        compiler_params=pltpu.CompilerParams(
            dimension_semantics=("arbitrary",)),
    )(a, b, bias.reshape(1, -1))


def _mm_stream_a(a, b, bias, relu, mt):
    M, K = a.shape
    _, N = b.shape
    return pl.pallas_call(
        functools.partial(_mm_kernel, relu=relu),
        grid=(M // mt,),
        in_specs=[
            pl.BlockSpec((mt, K), lambda i: (i, 0)),
            pl.BlockSpec((K, N), lambda i: (0, 0)),
            pl.BlockSpec((1, N), lambda i: (0, 0)),
        ],
        out_specs=pl.BlockSpec((mt, N), lambda i: (i, 0)),
        out_shape=jax.ShapeDtypeStruct((M, N), jnp.float32),
        compiler_params=pltpu.CompilerParams(
            dimension_semantics=("arbitrary",)),
    )(a, b, bias.reshape(1, -1))


def _dispatch_body(logits, slot0_ref, slot1_ref, wslot_ref, tok_ref,
                   cum_ref, a_ref):
    lane = jax.lax.broadcasted_iota(jnp.int32, (T, E), 1)
    m = jnp.max(logits, axis=1, keepdims=True)
    ex = jnp.exp(logits - m)
    probs = ex / jnp.sum(ex, axis=1, keepdims=True)

    p0 = jnp.max(probs, axis=1, keepdims=True)
    e0 = jnp.min(jnp.where(probs == p0, lane, E), axis=1, keepdims=True)
    pm = jnp.where(lane == e0, -1.0, probs)
    p1 = jnp.max(pm, axis=1, keepdims=True)
    e1 = jnp.min(jnp.where(pm == p1, lane, E), axis=1, keepdims=True)

    oh0 = (lane == e0).astype(jnp.float32)
    oh1 = (lane == e1).astype(jnp.float32)
    a_ref[...] = oh0 + oh1

    row = jax.lax.broadcasted_iota(jnp.int32, (128, 128), 0)
    col = jax.lax.broadcasted_iota(jnp.int32, (128, 128), 1)
    lstrict = (col < row).astype(jnp.float32)

    def body(i, carry):
        ablk = a_ref[pl.ds(i * 128, 128), :]
        cum_ref[pl.ds(i * 128, 128), :] = carry + _dot(lstrict, ablk)
        return carry + jnp.sum(ablk, axis=0, keepdims=True)

    jax.lax.fori_loop(0, T // 128, body, jnp.zeros((1, E), jnp.float32))
    cum = cum_ref[...]

    r0 = jnp.sum(cum * oh0, axis=1, keepdims=True)
    r1 = jnp.sum(cum * oh1, axis=1, keepdims=True)
    kept0 = r0 < float(CAP)
    kept1 = r1 < float(CAP)
    fs0 = e0.astype(jnp.float32) * STRIDE + r0
    fs1 = e1.astype(jnp.float32) * STRIDE + r1
    s0 = jnp.where(kept0, fs0, float(SENTINEL)).astype(jnp.int32)
    s1 = jnp.where(kept1, fs1, float(SENTINEL)).astype(jnp.int32)
    w0 = jnp.where(kept0, p0, 0.0)
    w1 = jnp.where(kept1, p1, 0.0)
    slot0_ref[...] = s0
    slot1_ref[...] = s1

    tf = jax.lax.broadcasted_iota(jnp.int32, (T, 1), 0).astype(jnp.float32)
    vt0 = jnp.where(kept0, tf, 0.0)
    vt1 = jnp.where(kept1, tf, 0.0)

    def wbody(j, _):
        sidx = j * 128 + jax.lax.broadcasted_iota(jnp.int32, (T, 128), 1)
        hit0 = s0 == sidx
        hit1 = s1 == sidx
        m0 = jnp.where(hit0, w0, 0.0)
        m1 = jnp.where(hit1, w1, 0.0)
        wslot_ref[:, pl.ds(j * 128, 128)] = jnp.sum(m0 + m1, axis=0,
                                                    keepdims=True)
        t0 = jnp.where(hit0, vt0, 0.0)
        t1 = jnp.where(hit1, vt1, 0.0)
        tok_ref[:, pl.ds(j * 128, 128)] = jnp.sum(t0 + t1, axis=0,
                                                  keepdims=True).astype(jnp.int32)
        return 0

    jax.lax.fori_loop(0, NSLOT // 128, wbody, 0)


def _dispatch_kernel(logits_ref, slot0_ref, slot1_ref, wslot_ref, tok_ref,
                     cum_ref, a_ref):
    _dispatch_body(logits_ref[...], slot0_ref, slot1_ref, wslot_ref, tok_ref,
                   cum_ref, a_ref)


def _dispatch(logits):
    return pl.pallas_call(
        _dispatch_kernel,
        in_specs=[pl.BlockSpec((T, E), lambda: (0, 0))],
        out_specs=[
            pl.BlockSpec((T, 1), lambda: (0, 0)),
            pl.BlockSpec((T, 1), lambda: (0, 0)),
            pl.BlockSpec((1, NSLOT), lambda: (0, 0)),
            pl.BlockSpec((1, NSLOT), lambda: (0, 0)),
        ],
        out_shape=[
            jax.ShapeDtypeStruct((T, 1), jnp.int32),
            jax.ShapeDtypeStruct((T, 1), jnp.int32),
            jax.ShapeDtypeStruct((1, NSLOT), jnp.float32),
            jax.ShapeDtypeStruct((1, NSLOT), jnp.int32),
        ],
        scratch_shapes=[pltpu.VMEM((T, E), jnp.float32),
                        pltpu.VMEM((T, E), jnp.float32)],
    )(logits)


def _mm2_kernel(a_ref, b_ref, bias_ref, wr3_ref, br3_ref, lg_ref, *, nsteps):
    j = pl.program_id(0)
    h2 = jnp.maximum(_dot(a_ref[...], b_ref[...]) + bias_ref[...], 0.0)
    part = _dot(h2, wr3_ref[...])

    @pl.when(j == 0)
    def _():
        lg_ref[...] = part + br3_ref[...]

    @pl.when(j > 0)
    def _():
        lg_ref[...] += part


def _mm2_logits(h1, Wr2, br2, Wr3, br3, nt=256):
    grid = (H // nt,)
    return pl.pallas_call(
        functools.partial(_mm2_kernel, nsteps=grid[0]),
        grid=grid,
        in_specs=[
            pl.BlockSpec((T, H), lambda j: (0, 0)),
            pl.BlockSpec((H, nt), lambda j: (0, j)),
            pl.BlockSpec((1, nt), lambda j: (0, j)),
            pl.BlockSpec((nt, E), lambda j: (j, 0)),
            pl.BlockSpec((1, E), lambda j: (0, 0)),
        ],
        out_specs=pl.BlockSpec((T, E), lambda j: (0, 0)),
        out_shape=jax.ShapeDtypeStruct((T, E), jnp.float32),
        compiler_params=pltpu.CompilerParams(
            dimension_semantics=("arbitrary",)),
    )(h1, Wr2, br2.reshape(1, -1), Wr3, br3.reshape(1, -1))


def _gather_kernel(s0_ref, s1_ref, x_ref, o_ref):
    e = pl.program_id(0)
    rows = e * STRIDE + jax.lax.broadcasted_iota(jnp.int32, (STRIDE, T), 0)
    sel = ((s0_ref[...] == rows).astype(jnp.float32)
           + (s1_ref[...] == rows).astype(jnp.float32))
    o_ref[...] = _dot(sel, x_ref[...])


def _gather(s0t, s1t, x2):
    return pl.pallas_call(
        _gather_kernel,
        grid=(E,),
        in_specs=[
            pl.BlockSpec((1, T), lambda e: (0, 0)),
            pl.BlockSpec((1, T), lambda e: (0, 0)),
            pl.BlockSpec((T, C), lambda e: (0, 0)),
        ],
        out_specs=pl.BlockSpec((STRIDE, C), lambda e: (e, 0)),
        out_shape=jax.ShapeDtypeStruct((NSLOT, C), jnp.float32),
        compiler_params=pltpu.CompilerParams(
            dimension_semantics=("arbitrary",)),
    )(s0t, s1t, x2)


def _combine_kernel(s0_ref, s1_ref, y_ref, o_ref, *, mt):
    scol = jax.lax.broadcasted_iota(jnp.int32, (mt, NSLOT), 1)
    sel = ((s0_ref[...] == scol).astype(jnp.float32)
           + (s1_ref[...] == scol).astype(jnp.float32))
    o_ref[...] = _dot(sel, y_ref[...])


def _combine(s0, s1, Y, mt=256):
    return pl.pallas_call(
        functools.partial(_combine_kernel, mt=mt),
        grid=(T // mt,),
        in_specs=[
            pl.BlockSpec((mt, 1), lambda i: (i, 0)),
            pl.BlockSpec((mt, 1), lambda i: (i, 0)),
            pl.BlockSpec((NSLOT, C), lambda i: (0, 0)),
        ],
        out_specs=pl.BlockSpec((mt, C), lambda i: (i, 0)),
        out_shape=jax.ShapeDtypeStruct((T, C), jnp.float32),
        compiler_params=pltpu.CompilerParams(
            dimension_semantics=("arbitrary",)),
    )(s0, s1, Y)


_SC_MESH = plsc.VectorSubcoreMesh(core_axis_name="c", subcore_axis_name="s")


@functools.partial(
    pl.kernel, mesh=_SC_MESH,
    out_type=jax.ShapeDtypeStruct((NSLOT, C), jnp.float32),
    scratch_types=[
        pltpu.VMEM((SLOT_PER_W,), jnp.int32),
        pltpu.VMEM((SLOT_PER_W, C), jnp.float32),
        pltpu.SemaphoreType.DMA,
    ],
)
def _sc_gather(x_hbm, tok_hbm, xe_hbm, idx_v, rows_v, sem):
    wid = lax.axis_index("s") * NC + lax.axis_index("c")
    base = wid * SLOT_PER_W
    pltpu.sync_copy(tok_hbm.at[pl.ds(base, SLOT_PER_W)], idx_v)
    nchunk, csz = SLOT_PER_W // 8, 8
    copies = [
        pltpu.async_copy(x_hbm.at[idx_v.at[pl.ds(i * csz, csz)]],
                         rows_v.at[pl.ds(i * csz, csz)], sem)
        for i in range(nchunk)
    ]
    for cp in copies:
        cp.wait()
    pltpu.sync_copy(rows_v, xe_hbm.at[pl.ds(base, SLOT_PER_W)])


def _ffn_kernel(xe_ref, w1_ref, b1_ref, w2_ref, b2_ref, ws_ref, o_ref,
                *, nsteps):
    j = pl.program_id(1)
    hblk = jnp.maximum(_dot(xe_ref[...], w1_ref[0]) + b1_ref[0], 0.0)
    part = _dot(hblk, w2_ref[0])

    @pl.when(j == 0)
    def _():
        o_ref[...] = part

    @pl.when(j > 0)
    def _():
        o_ref[...] += part

    @pl.when(j == nsteps - 1)
    def _():
        o_ref[...] = (o_ref[...] + b2_ref[0]) * ws_ref[...]


def _ffn(xe, W1, b1, W2, b2, wslot_col, ht=1024):
    grid = (E, H // ht)
    return pl.pallas_call(
        functools.partial(_ffn_kernel, nsteps=grid[1]),
        grid=grid,
        in_specs=[
            pl.BlockSpec((STRIDE, C), lambda e, j: (e, 0)),
            pl.BlockSpec((1, C, ht), lambda e, j: (e, 0, j)),
            pl.BlockSpec((1, 1, ht), lambda e, j: (e, 0, j)),
            pl.BlockSpec((1, ht, C), lambda e, j: (e, j, 0)),
            pl.BlockSpec((1, 1, C), lambda e, j: (e, 0, 0)),
            pl.BlockSpec((STRIDE, 1), lambda e, j: (e, 0)),
        ],
        out_specs=pl.BlockSpec((STRIDE, C), lambda e, j: (e, 0)),
        out_shape=jax.ShapeDtypeStruct((NSLOT, C), jnp.float32),
        compiler_params=pltpu.CompilerParams(
            dimension_semantics=("arbitrary", "arbitrary")),
    )(xe, W1, b1.reshape(E, 1, H), W2, b2.reshape(E, 1, C), wslot_col)


@functools.partial(
    pl.kernel, mesh=_SC_MESH,
    out_type=jax.ShapeDtypeStruct((T, C), jnp.float32),
    scratch_types=[
        pltpu.VMEM((32,), jnp.int32),
        pltpu.VMEM((32,), jnp.int32),
        pltpu.VMEM((32, C), jnp.float32),
        pltpu.VMEM((32, C), jnp.float32),
        pltpu.SemaphoreType.DMA,
    ],
)
def _sc_combine(y_hbm, s0_hbm, s1_hbm, out_hbm, i0_v, i1_v, r0_v, r1_v, sem):
    wid = lax.axis_index("s") * NC + lax.axis_index("c")
    base = wid * TOK_PER_W
    for half in range(2):
        off = base + half * 32
        pltpu.sync_copy(s0_hbm.at[pl.ds(off, 32)], i0_v)
        pltpu.sync_copy(s1_hbm.at[pl.ds(off, 32)], i1_v)
        copies = [
            pltpu.async_copy(y_hbm.at[iv.at[pl.ds(i * 8, 8)]],
                             rv.at[pl.ds(i * 8, 8)], sem)
            for i in range(4) for iv, rv in ((i0_v, r0_v), (i1_v, r1_v))
        ]
        for cp in copies:
            cp.wait()
        for r in range(32):
            def add_body(cc, _):
                sl = pl.ds(cc * 16, 16)
                r0_v[r, sl] = r0_v[r, sl] + r1_v[r, sl]
                return 0
            lax.fori_loop(0, C // 16, add_body, 0, unroll=8)
        pltpu.sync_copy(r0_v, out_hbm.at[pl.ds(off, 32)])


def kernel(x, Wr1, br1, Wr2, br2, Wr3, br3, W1, b1, W2, b2):
    x2 = x.reshape(T, C)
    h1 = _mm_resident(x2, Wr1, br1, True, 512)
    logits = _mm2_logits(h1, Wr2, br2, Wr3, br3)
    slot0, slot1, wslot, tok = _dispatch(logits)
    xe = _gather(slot0.reshape(1, T), slot1.reshape(1, T), x2)
    Y = _ffn(xe, W1, b1, W2, b2, wslot.reshape(NSLOT, 1))
    out = _combine(slot0, slot1, Y)
    return out.reshape(1, T, C)


# speedup vs baseline: 1.6480x; 1.0217x over previous
import functools

import jax
import jax.numpy as jnp
from jax.experimental import pallas as pl
from jax.experimental.pallas import tpu as pltpu

T, C, H = 2048, 1024, 4096
E, TOPK = 8, 2
CAP = 320
STRIDE = 336
NSLOT = E * STRIDE
SENTINEL = CAP


def _dot(a, b):
    return jax.lax.dot_general(a, b, (((1,), (0,)), ((), ())),
                               preferred_element_type=jnp.float32)


def _mm_kernel(a_ref, b_ref, bias_ref, o_ref, *, relu):
    acc = _dot(a_ref[...], b_ref[...]) + bias_ref[...]
    o_ref[...] = jnp.maximum(acc, 0.0) if relu else acc


def _mm_resident(a, b, bias, relu, nt):
    M, K = a.shape
    _, N = b.shape
    return pl.pallas_call(
        functools.partial(_mm_kernel, relu=relu),
        grid=(N // nt,),
        in_specs=[
            pl.BlockSpec((M, K), lambda j: (0, 0)),
            pl.BlockSpec((K, nt), lambda j: (0, j)),
            pl.BlockSpec((1, nt), lambda j: (0, j)),
        ],
        out_specs=pl.BlockSpec((M, nt), lambda j: (0, j)),
        out_shape=jax.ShapeDtypeStruct((M, N), jnp.float32),
        compiler_params=pltpu.CompilerParams(
            dimension_semantics=("arbitrary",)),
    )(a, b, bias.reshape(1, -1))


def _dispatch_body(logits, slot0_ref, slot1_ref, wslot_ref, cum_ref, a_ref):
    lane = jax.lax.broadcasted_iota(jnp.int32, (T, E), 1)
    m = jnp.max(logits, axis=1, keepdims=True)
    ex = jnp.exp(logits - m)
    probs = ex / jnp.sum(ex, axis=1, keepdims=True)

    p0 = jnp.max(probs, axis=1, keepdims=True)
    e0 = jnp.min(jnp.where(probs == p0, lane, E), axis=1, keepdims=True)
    pm = jnp.where(lane == e0, -1.0, probs)
    p1 = jnp.max(pm, axis=1, keepdims=True)
    e1 = jnp.min(jnp.where(pm == p1, lane, E), axis=1, keepdims=True)

    oh0 = (lane == e0).astype(jnp.float32)
    oh1 = (lane == e1).astype(jnp.float32)
    a_ref[...] = oh0 + oh1

    row = jax.lax.broadcasted_iota(jnp.int32, (128, 128), 0)
    col = jax.lax.broadcasted_iota(jnp.int32, (128, 128), 1)
    lstrict = (col < row).astype(jnp.float32)

    def body(i, carry):
        ablk = a_ref[pl.ds(i * 128, 128), :]
        cum_ref[pl.ds(i * 128, 128), :] = carry + _dot(lstrict, ablk)
        return carry + jnp.sum(ablk, axis=0, keepdims=True)

    jax.lax.fori_loop(0, T // 128, body, jnp.zeros((1, E), jnp.float32))
    cum = cum_ref[...]

    r0 = jnp.sum(cum * oh0, axis=1, keepdims=True)
    r1 = jnp.sum(cum * oh1, axis=1, keepdims=True)
    kept0 = r0 < float(CAP)
    kept1 = r1 < float(CAP)
    fs0 = e0.astype(jnp.float32) * STRIDE + r0
    fs1 = e1.astype(jnp.float32) * STRIDE + r1
    s0 = jnp.where(kept0, fs0, float(SENTINEL)).astype(jnp.int32)
    s1 = jnp.where(kept1, fs1, float(SENTINEL)).astype(jnp.int32)
    w0 = jnp.where(kept0, p0, 0.0)
    w1 = jnp.where(kept1, p1, 0.0)
    slot0_ref[...] = s0
    slot1_ref[...] = s1

    def wbody(j, _):
        sidx = j * 128 + jax.lax.broadcasted_iota(jnp.int32, (T, 128), 1)
        m0 = jnp.where(s0 == sidx, w0, 0.0)
        m1 = jnp.where(s1 == sidx, w1, 0.0)
        wslot_ref[:, pl.ds(j * 128, 128)] = jnp.sum(m0 + m1, axis=0,
                                                    keepdims=True)
        return 0

    jax.lax.fori_loop(0, NSLOT // 128, wbody, 0)


def _dispatch_kernel(logits_ref, slot0_ref, slot1_ref, wslot_ref,
                     cum_ref, a_ref):
    _dispatch_body(logits_ref[...], slot0_ref, slot1_ref, wslot_ref,
                   cum_ref, a_ref)


def _dispatch(logits):
    return pl.pallas_call(
        _dispatch_kernel,
        in_specs=[pl.BlockSpec((T, E), lambda: (0, 0))],
        out_specs=[
            pl.BlockSpec((T, 1), lambda: (0, 0)),
            pl.BlockSpec((T, 1), lambda: (0, 0)),
            pl.BlockSpec((1, NSLOT), lambda: (0, 0)),
        ],
        out_shape=[
            jax.ShapeDtypeStruct((T, 1), jnp.int32),
            jax.ShapeDtypeStruct((T, 1), jnp.int32),
            jax.ShapeDtypeStruct((1, NSLOT), jnp.float32),
        ],
        scratch_shapes=[pltpu.VMEM((T, E), jnp.float32),
                        pltpu.VMEM((T, E), jnp.float32)],
    )(logits)


def _mm2_kernel(a_ref, b_ref, bias_ref, wr3_ref, br3_ref, lg_ref, *, nsteps):
    j = pl.program_id(0)
    h2 = jnp.maximum(_dot(a_ref[...], b_ref[...]) + bias_ref[...], 0.0)
    part = _dot(h2, wr3_ref[...])

    @pl.when(j == 0)
    def _():
        lg_ref[...] = part + br3_ref[...]

    @pl.when(j > 0)
    def _():
        lg_ref[...] += part


def _mm2_logits(h1, Wr2, br2, Wr3, br3, nt=256):
    grid = (H // nt,)
    return pl.pallas_call(
        functools.partial(_mm2_kernel, nsteps=grid[0]),
        grid=grid,
        in_specs=[
            pl.BlockSpec((T, H), lambda j: (0, 0)),
            pl.BlockSpec((H, nt), lambda j: (0, j)),
            pl.BlockSpec((1, nt), lambda j: (0, j)),
            pl.BlockSpec((nt, E), lambda j: (j, 0)),
            pl.BlockSpec((1, E), lambda j: (0, 0)),
        ],
        out_specs=pl.BlockSpec((T, E), lambda j: (0, 0)),
        out_shape=jax.ShapeDtypeStruct((T, E), jnp.float32),
        compiler_params=pltpu.CompilerParams(
            dimension_semantics=("arbitrary",)),
    )(h1, Wr2, br2.reshape(1, -1), Wr3, br3.reshape(1, -1))


def _gather_kernel(s0_ref, s1_ref, x_ref, o_ref):
    e = pl.program_id(0)
    rows = e * STRIDE + jax.lax.broadcasted_iota(jnp.int32, (STRIDE, T), 0)
    sel = ((s0_ref[...] == rows).astype(jnp.float32)
           + (s1_ref[...] == rows).astype(jnp.float32))
    o_ref[...] = _dot(sel, x_ref[...])


def _gather(s0t, s1t, x2):
    return pl.pallas_call(
        _gather_kernel,
        grid=(E,),
        in_specs=[
            pl.BlockSpec((1, T), lambda e: (0, 0)),
            pl.BlockSpec((1, T), lambda e: (0, 0)),
            pl.BlockSpec((T, C), lambda e: (0, 0)),
        ],
        out_specs=pl.BlockSpec((STRIDE, C), lambda e: (e, 0)),
        out_shape=jax.ShapeDtypeStruct((NSLOT, C), jnp.float32),
        compiler_params=pltpu.CompilerParams(
            dimension_semantics=("arbitrary",)),
    )(s0t, s1t, x2)


def _combine_kernel(s0_ref, s1_ref, y_ref, o_ref, *, mt):
    scol = jax.lax.broadcasted_iota(jnp.int32, (mt, NSLOT), 1)
    sel = ((s0_ref[...] == scol).astype(jnp.float32)
           + (s1_ref[...] == scol).astype(jnp.float32))
    o_ref[...] = _dot(sel, y_ref[...])


def _combine(s0, s1, Y, mt=256):
    return pl.pallas_call(
        functools.partial(_combine_kernel, mt=mt),
        grid=(T // mt,),
        in_specs=[
            pl.BlockSpec((mt, 1), lambda i: (i, 0)),
            pl.BlockSpec((mt, 1), lambda i: (i, 0)),
            pl.BlockSpec((NSLOT, C), lambda i: (0, 0)),
        ],
        out_specs=pl.BlockSpec((mt, C), lambda i: (i, 0)),
        out_shape=jax.ShapeDtypeStruct((T, C), jnp.float32),
        compiler_params=pltpu.CompilerParams(
            dimension_semantics=("arbitrary",)),
    )(s0, s1, Y)


def _ffn_kernel(xe_ref, w1_ref, b1_ref, w2_ref, b2_ref, ws_ref, o_ref,
                *, nsteps):
    j = pl.program_id(1)
    hblk = jnp.maximum(_dot(xe_ref[...], w1_ref[0]) + b1_ref[0], 0.0)
    part = _dot(hblk, w2_ref[0])

    @pl.when(j == 0)
    def _():
        o_ref[...] = part

    @pl.when(j > 0)
    def _():
        o_ref[...] += part

    @pl.when(j == nsteps - 1)
    def _():
        o_ref[...] = (o_ref[...] + b2_ref[0]) * ws_ref[...]


def _ffn(xe, W1, b1, W2, b2, wslot_col, ht=1024):
    grid = (E, H // ht)
    return pl.pallas_call(
        functools.partial(_ffn_kernel, nsteps=grid[1]),
        grid=grid,
        in_specs=[
            pl.BlockSpec((STRIDE, C), lambda e, j: (e, 0)),
            pl.BlockSpec((1, C, ht), lambda e, j: (e, 0, j)),
            pl.BlockSpec((1, 1, ht), lambda e, j: (e, 0, j)),
            pl.BlockSpec((1, ht, C), lambda e, j: (e, j, 0)),
            pl.BlockSpec((1, 1, C), lambda e, j: (e, 0, 0)),
            pl.BlockSpec((STRIDE, 1), lambda e, j: (e, 0)),
        ],
        out_specs=pl.BlockSpec((STRIDE, C), lambda e, j: (e, 0)),
        out_shape=jax.ShapeDtypeStruct((NSLOT, C), jnp.float32),
        compiler_params=pltpu.CompilerParams(
            dimension_semantics=("arbitrary", "arbitrary")),
    )(xe, W1, b1.reshape(E, 1, H), W2, b2.reshape(E, 1, C), wslot_col)


def kernel(x, Wr1, br1, Wr2, br2, Wr3, br3, W1, b1, W2, b2):
    x2 = x.reshape(T, C)
    h1 = _mm_resident(x2, Wr1, br1, True, 512)
    logits = _mm2_logits(h1, Wr2, br2, Wr3, br3)
    slot0, slot1, wslot = _dispatch(logits)
    xe = _gather(slot0.reshape(1, T), slot1.reshape(1, T), x2)
    Y = _ffn(xe, W1, b1, W2, b2, wslot.reshape(NSLOT, 1))
    out = _combine(slot0, slot1, Y)
    return out.reshape(1, T, C)
